# Initial kernel scaffold; baseline (speedup 1.0000x reference)
import jax
import jax.numpy as jnp
from jax.experimental import pallas as pl


def kernel(pred):
    raise NotImplementedError("write your pallas kernel here")


# speedup vs baseline: 34.6808x; 34.6808x over previous
import jax
import jax.numpy as jnp
from jax import lax
from jax.experimental import pallas as pl
from jax.experimental.pallas import tpu as pltpu

_B = 32
_H = 1024
_W = 1024
_N = _H * _W
_K = 838860
_EPS = 1e-06


def _lcl_batch_kernel(x_ref, out_ref, l_ref):
    x = x_ref[0]
    z_row = jnp.zeros((1, _W), jnp.float32)
    z_col = jnp.zeros((_H, 1), jnp.float32)
    up = jnp.concatenate([x[1:, :], z_row], axis=0)
    down = jnp.concatenate([z_row, x[:-1, :]], axis=0)
    left = jnp.concatenate([x[:, 1:], z_col], axis=1)
    right = jnp.concatenate([z_col, x[:, :-1]], axis=1)
    l_ref[...] = jnp.abs(up + down + left + right - 4.0 * x)

    def body(i, prefix):
        bit = 30 - i
        cand = prefix | (jnp.int32(1) << bit)
        xi = lax.bitcast_convert_type(l_ref[...], jnp.int32)
        cnt = jnp.sum((xi < cand).astype(jnp.float32))
        return lax.select(cnt <= jnp.float32(_K), cand, prefix)

    v = lax.fori_loop(0, 31, body, jnp.int32(0))

    L = l_ref[...]
    xi = lax.bitcast_convert_type(L, jnp.int32)
    maskf = (xi > v).astype(jnp.float32)
    edge_sum = jnp.sum(L * maskf)
    edge_cnt = jnp.sum(maskf)
    flat_sum = jnp.sum(L * (1.0 - maskf))

    lane = lax.broadcasted_iota(jnp.int32, (1, 1, 128), 2)
    vec = jnp.where(lane == 0, edge_sum,
          jnp.where(lane == 1, edge_cnt,
          jnp.where(lane == 2, flat_sum, 0.0)))
    out_ref[...] = vec


def _combine_kernel(p_ref, out_ref):
    p = p_ref[...]
    es = jnp.sum(p[:, :, 0:1])
    ec = jnp.sum(p[:, :, 1:2])
    fs = jnp.sum(p[:, :, 2:3])
    fc = jnp.float32(_B * _N) - ec
    edge_mean = jnp.where(ec > 0, es / jnp.maximum(ec, 1.0), 0.0)
    flat_mean = jnp.where(fc > 0, fs / jnp.maximum(fc, 1.0), jnp.float32(_EPS))
    result = flat_mean / (edge_mean + jnp.float32(_EPS))
    out_ref[...] = jnp.broadcast_to(result, (1, 128))


def kernel(pred):
    x = pred.reshape(_B, _H, _W)
    partials = pl.pallas_call(
        _lcl_batch_kernel,
        out_shape=jax.ShapeDtypeStruct((_B, 1, 128), jnp.float32),
        grid=(_B,),
        in_specs=[pl.BlockSpec((1, _H, _W), lambda b: (b, 0, 0))],
        out_specs=pl.BlockSpec((1, 1, 128), lambda b: (b, 0, 0)),
        scratch_shapes=[pltpu.VMEM((_H, _W), jnp.float32)],
        compiler_params=pltpu.CompilerParams(
            dimension_semantics=("parallel",),
        ),
        name="lcl_batch",
    )(x)
    out = pl.pallas_call(
        _combine_kernel,
        out_shape=jax.ShapeDtypeStruct((1, 128), jnp.float32),
        name="lcl_combine",
    )(partials)
    return out[0, 0]


# speedup vs baseline: 57.9581x; 1.6712x over previous
import jax
import jax.numpy as jnp
from jax import lax
from jax.experimental import pallas as pl
from jax.experimental.pallas import tpu as pltpu

_B = 32
_H = 1024
_W = 1024
_N = _H * _W
_K = 838860
_EPS = 1e-06


def _lcl_batch_kernel(x_ref, out_ref, l_ref):
    x = x_ref[0]
    z_row = jnp.zeros((1, _W), jnp.float32)
    z_col = jnp.zeros((_H, 1), jnp.float32)
    up = jnp.concatenate([x[1:, :], z_row], axis=0)
    down = jnp.concatenate([z_row, x[:-1, :]], axis=0)
    left = jnp.concatenate([x[:, 1:], z_col], axis=1)
    right = jnp.concatenate([z_col, x[:, :-1]], axis=1)
    l_ref[...] = jnp.abs(up + down + left + right - 4.0 * x)

    def body(i, prefix):
        bit = 30 - i
        cand = prefix | (jnp.int32(1) << bit)
        xi = lax.bitcast_convert_type(l_ref[...], jnp.int32)
        m = (xi < cand).astype(jnp.float32)
        s1 = jnp.sum(m.reshape(8, 128, _W), axis=0)
        s2 = jnp.sum(s1.reshape(8, 16, _W), axis=0)
        cnt = jnp.sum(s2)
        return lax.select(cnt <= jnp.float32(_K), cand, prefix)

    v = lax.fori_loop(0, 31, body, jnp.int32(0))

    L = l_ref[...]
    xi = lax.bitcast_convert_type(L, jnp.int32)
    maskf = (xi > v).astype(jnp.float32)
    edge_sum = jnp.sum(L * maskf)
    edge_cnt = jnp.sum(maskf)
    flat_sum = jnp.sum(L * (1.0 - maskf))

    lane = lax.broadcasted_iota(jnp.int32, (1, 1, 128), 2)
    vec = jnp.where(lane == 0, edge_sum,
          jnp.where(lane == 1, edge_cnt,
          jnp.where(lane == 2, flat_sum, 0.0)))
    out_ref[...] = vec


def _combine_kernel(p_ref, out_ref):
    p = p_ref[...]
    es = jnp.sum(p[:, :, 0:1])
    ec = jnp.sum(p[:, :, 1:2])
    fs = jnp.sum(p[:, :, 2:3])
    fc = jnp.float32(_B * _N) - ec
    edge_mean = jnp.where(ec > 0, es / jnp.maximum(ec, 1.0), 0.0)
    flat_mean = jnp.where(fc > 0, fs / jnp.maximum(fc, 1.0), jnp.float32(_EPS))
    result = flat_mean / (edge_mean + jnp.float32(_EPS))
    out_ref[...] = jnp.broadcast_to(result, (1, 128))


def kernel(pred):
    x = pred.reshape(_B, _H, _W)
    partials = pl.pallas_call(
        _lcl_batch_kernel,
        out_shape=jax.ShapeDtypeStruct((_B, 1, 128), jnp.float32),
        grid=(_B,),
        in_specs=[pl.BlockSpec((1, _H, _W), lambda b: (b, 0, 0))],
        out_specs=pl.BlockSpec((1, 1, 128), lambda b: (b, 0, 0)),
        scratch_shapes=[pltpu.VMEM((_H, _W), jnp.float32)],
        compiler_params=pltpu.CompilerParams(
            dimension_semantics=("parallel",),
        ),
        name="lcl_batch",
    )(x)
    out = pl.pallas_call(
        _combine_kernel,
        out_shape=jax.ShapeDtypeStruct((1, 128), jnp.float32),
        name="lcl_combine",
    )(partials)
    return out[0, 0]
